# Initial kernel scaffold; baseline (speedup 1.0000x reference)
import jax
import jax.numpy as jnp
from jax.experimental import pallas as pl


def kernel(x, edge_index, edge_weight, W1, b1, W2, b2):
    raise NotImplementedError("write your pallas kernel here")


# speedup vs baseline: 4.1764x; 4.1764x over previous
import functools

import jax
import jax.numpy as jnp
from jax import lax
from jax.experimental import pallas as pl
from jax.experimental.pallas import tpu as pltpu
from jax.experimental.pallas import tpu_sc as plsc

NC = 2
NS = 16
NW = NC * NS
CH = 80


def _sc_gather_scatter(h, src, dst, ew, zeros):
    n, d = h.shape
    e = src.shape[0]
    epw = e // NW
    nit = epw // CH
    slab = (n // NS) // 8 * 8
    tail = n - NS * slab
    mesh = plsc.VectorSubcoreMesh(core_axis_name="c", subcore_axis_name="s")

    @functools.partial(
        pl.kernel,
        out_type=jax.ShapeDtypeStruct((NC, n, d), jnp.float32),
        mesh=mesh,
        scratch_types=[
            pltpu.VMEM((CH,), jnp.int32),
            pltpu.VMEM((CH,), jnp.int32),
            pltpu.VMEM((CH,), jnp.float32),
            pltpu.VMEM((CH, d), jnp.float32),
            pltpu.VMEM_SHARED((n, d), jnp.float32),
            pltpu.SemaphoreType.DMA,
        ],
    )
    def body(h_hbm, src_hbm, dst_hbm, ew_hbm, z_hbm, out_hbm,
             si, di, wv, rows, acc, sem):
        c = lax.axis_index("c")
        s = lax.axis_index("s")
        wid = c * NS + s
        pltpu.sync_copy(z_hbm.at[pl.ds(s * slab, slab)],
                        acc.at[pl.ds(s * slab, slab)])

        @pl.when(s == NS - 1)
        def _():
            pltpu.sync_copy(z_hbm.at[pl.ds(NS * slab, tail)],
                            acc.at[pl.ds(NS * slab, tail)])

        plsc.subcore_barrier()

        base0 = pl.multiple_of(wid * epw, 8)

        def chunk(it, carry):
            base = pl.multiple_of(base0 + it * CH, 8)
            pltpu.sync_copy(src_hbm.at[pl.ds(base, CH)], si)
            pltpu.sync_copy(dst_hbm.at[pl.ds(base, CH)], di)
            pltpu.sync_copy(ew_hbm.at[pl.ds(base, CH)], wv)
            pltpu.async_copy(h_hbm.at[si], rows, sem).wait()

            def scale(g, carry2):
                w16 = wv[pl.ds(g * 16, 16)]
                for i in range(16):
                    e_i = g * 16 + i
                    w = w16[i]
                    for j in range(d // 16):
                        sl = pl.ds(j * 16, 16)
                        rows[e_i, sl] = rows[e_i, sl] * w
                return carry2

            lax.fori_loop(0, CH // 16, scale, 0)
            pltpu.sync_copy(rows, acc.at[di], add=True)
            return carry

        lax.fori_loop(0, nit, chunk, 0)
        plsc.subcore_barrier()
        pltpu.sync_copy(acc.at[pl.ds(s * slab, slab)],
                        out_hbm.at[c, pl.ds(s * slab, slab)])

        @pl.when(s == NS - 1)
        def _():
            pltpu.sync_copy(acc.at[pl.ds(NS * slab, tail)],
                            out_hbm.at[c, pl.ds(NS * slab, tail)])

    return body(h, src, dst, ew, zeros)


def _linear_relu_tc(p, W, b):
    k, n, d = p.shape
    br = 1000

    def body(p_ref, w_ref, b_ref, o_ref):
        xs = jnp.sum(p_ref[...], axis=0)
        y = jnp.dot(xs, w_ref[...], preferred_element_type=jnp.float32)
        o_ref[...] = jnp.maximum(y + b_ref[...], 0.0)

    return pl.pallas_call(
        body,
        grid=(n // br,),
        in_specs=[
            pl.BlockSpec((k, br, d), lambda i: (0, i, 0)),
            pl.BlockSpec((d, d), lambda i: (0, 0)),
            pl.BlockSpec((1, d), lambda i: (0, 0)),
        ],
        out_specs=pl.BlockSpec((br, d), lambda i: (i, 0)),
        out_shape=jax.ShapeDtypeStruct((n, d), jnp.float32),
    )(p, W, b.reshape(1, d))


def _sum_partials_tc(p):
    k, n, d = p.shape
    br = 1000

    def body(p_ref, o_ref):
        o_ref[...] = jnp.sum(p_ref[...], axis=0)

    return pl.pallas_call(
        body,
        grid=(n // br,),
        in_specs=[pl.BlockSpec((k, br, d), lambda i: (0, i, 0))],
        out_specs=pl.BlockSpec((br, d), lambda i: (i, 0)),
        out_shape=jax.ShapeDtypeStruct((n, d), jnp.float32),
    )(p)


def kernel(x, edge_index, edge_weight, W1, b1, W2, b2):
    src = edge_index[0]
    dst = edge_index[1]
    zeros = jnp.zeros(x.shape, jnp.float32)
    h1 = _linear_relu_tc(x[None], W1, b1)
    p1 = _sc_gather_scatter(h1, src, dst, edge_weight, zeros)
    h2 = _linear_relu_tc(p1, W2, b2)
    p2 = _sc_gather_scatter(h2, src, dst, edge_weight, zeros)
    return _sum_partials_tc(p2)


# speedup vs baseline: 5.0508x; 1.2094x over previous
import functools

import jax
import jax.numpy as jnp
from jax import lax
from jax.experimental import pallas as pl
from jax.experimental.pallas import tpu as pltpu
from jax.experimental.pallas import tpu_sc as plsc

NC = 2
NS = 16
NW = NC * NS
CH = 128


def _sc_gather_scatter(h, pk, zeros):
    n, d = h.shape
    nw, nit, _, _ = pk.shape
    slab = (n // NS) // 8 * 8
    tail = n - NS * slab
    mesh = plsc.VectorSubcoreMesh(core_axis_name="c", subcore_axis_name="s")

    @functools.partial(
        pl.kernel,
        out_type=jax.ShapeDtypeStruct((NC, n, d), jnp.float32),
        mesh=mesh,
        scratch_types=[
            pltpu.VMEM((3, CH), jnp.int32),
            pltpu.VMEM((3, CH), jnp.int32),
            pltpu.VMEM((CH, d), jnp.float32),
            pltpu.VMEM((CH, d), jnp.float32),
            pltpu.VMEM_SHARED((n, d), jnp.float32),
            pltpu.SemaphoreType.DMA,
            pltpu.SemaphoreType.DMA,
            pltpu.SemaphoreType.DMA,
            pltpu.SemaphoreType.DMA,
        ],
    )
    def body(h_hbm, pk_hbm, z_hbm, out_hbm,
             pk0, pk1, r0, r1, acc, ps0, ps1, gs0, gs1):
        c = lax.axis_index("c")
        s = lax.axis_index("s")
        wid = c * NS + s
        pltpu.sync_copy(z_hbm.at[pl.ds(s * slab, slab)],
                        acc.at[pl.ds(s * slab, slab)])

        @pl.when(s == NS - 1)
        def _():
            pltpu.sync_copy(z_hbm.at[pl.ds(NS * slab, tail)],
                            acc.at[pl.ds(NS * slab, tail)])

        plsc.subcore_barrier()

        pltpu.sync_copy(pk_hbm.at[wid, 0], pk0)
        pltpu.async_copy(h_hbm.at[pk0.at[0]], r0, gs0)
        pltpu.async_copy(pk_hbm.at[wid, 1], pk1, ps1)

        def process(it, pkc, psem, rows, gsem, opk, opsem, orows, ogsem):
            pltpu.make_async_copy(h_hbm.at[pkc.at[0]], rows, gsem).wait()

            @pl.when(it < nit - 1)
            def _():
                pltpu.make_async_copy(pk_hbm.at[wid, it + 1], opk,
                                      opsem).wait()
                pltpu.async_copy(h_hbm.at[opk.at[0]], orows, ogsem)

            def scale(g, carry):
                wbits = pkc[2, pl.ds(g * 16, 16)]
                w16 = lax.bitcast_convert_type(wbits, jnp.float32)
                for i in range(16):
                    e_i = g * 16 + i
                    w = w16[i]
                    for j in range(d // 16):
                        sl = pl.ds(j * 16, 16)
                        rows[e_i, sl] = rows[e_i, sl] * w
                return carry

            lax.fori_loop(0, CH // 16, scale, 0)

            pltpu.sync_copy(rows, acc.at[pkc.at[1]], add=True)

            @pl.when(it < nit - 2)
            def _():
                pltpu.async_copy(pk_hbm.at[wid, it + 2], pkc, psem)

        def process_even(it):
            process(it, pk0, ps0, r0, gs0, pk1, ps1, r1, gs1)

        def process_odd(it):
            process(it, pk1, ps1, r1, gs1, pk0, ps0, r0, gs0)

        def pair(it2, carry):
            process_even(it2 * 2)
            process_odd(it2 * 2 + 1)
            return carry

        lax.fori_loop(0, nit // 2, pair, 0)
        if nit % 2:
            process_even(nit - 1)
        plsc.subcore_barrier()
        pltpu.sync_copy(acc.at[pl.ds(s * slab, slab)],
                        out_hbm.at[c, pl.ds(s * slab, slab)])

        @pl.when(s == NS - 1)
        def _():
            pltpu.sync_copy(acc.at[pl.ds(NS * slab, tail)],
                            out_hbm.at[c, pl.ds(NS * slab, tail)])

    return body(h, pk, zeros)


def _linear_relu_tc(p, W, b):
    k, n, d = p.shape
    br = 1000

    def body(p_ref, w_ref, b_ref, o_ref):
        xs = jnp.sum(p_ref[...], axis=0)
        y = jnp.dot(xs, w_ref[...], preferred_element_type=jnp.float32)
        o_ref[...] = jnp.maximum(y + b_ref[...], 0.0)

    return pl.pallas_call(
        body,
        grid=(n // br,),
        in_specs=[
            pl.BlockSpec((k, br, d), lambda i: (0, i, 0)),
            pl.BlockSpec((d, d), lambda i: (0, 0)),
            pl.BlockSpec((1, d), lambda i: (0, 0)),
        ],
        out_specs=pl.BlockSpec((br, d), lambda i: (i, 0)),
        out_shape=jax.ShapeDtypeStruct((n, d), jnp.float32),
    )(p, W, b.reshape(1, d))


def _sum_partials_tc(p):
    k, n, d = p.shape
    br = 1000

    def body(p_ref, o_ref):
        o_ref[...] = jnp.sum(p_ref[...], axis=0)

    return pl.pallas_call(
        body,
        grid=(n // br,),
        in_specs=[pl.BlockSpec((k, br, d), lambda i: (0, i, 0))],
        out_specs=pl.BlockSpec((br, d), lambda i: (i, 0)),
        out_shape=jax.ShapeDtypeStruct((n, d), jnp.float32),
    )(p)


def kernel(x, edge_index, edge_weight, W1, b1, W2, b2):
    e = edge_weight.shape[0]
    epw_pad = -(-e // (NW * CH)) * CH
    nit = epw_pad // CH
    e_pad = NW * epw_pad
    src = jnp.pad(edge_index[0], (0, e_pad - e))
    dst = jnp.pad(edge_index[1], (0, e_pad - e))
    ewb = jnp.pad(edge_weight, (0, e_pad - e)).view(jnp.int32)
    pk = jnp.stack([src, dst, ewb], axis=1).reshape(NW, nit, CH, 3)
    pk = jnp.swapaxes(pk, 2, 3)
    zeros = jnp.zeros(x.shape, jnp.float32)
    h1 = _linear_relu_tc(x[None], W1, b1)
    p1 = _sc_gather_scatter(h1, pk, zeros)
    h2 = _linear_relu_tc(p1, W2, b2)
    p2 = _sc_gather_scatter(h2, pk, zeros)
    return _sum_partials_tc(p2)


# speedup vs baseline: 5.3572x; 1.0607x over previous
import functools

import jax
import jax.numpy as jnp
from jax import lax
from jax.experimental import pallas as pl
from jax.experimental.pallas import tpu as pltpu
from jax.experimental.pallas import tpu_sc as plsc

NC = 2
NS = 16
NW = NC * NS
CH = 128


def _sc_gather_scatter(h, pk, dst3, zeros):
    n, d = h.shape
    nw, nit, _, _ = pk.shape
    slab = (n // NS) // 8 * 8
    tail = n - NS * slab
    mesh = plsc.VectorSubcoreMesh(core_axis_name="c", subcore_axis_name="s")

    @functools.partial(
        pl.kernel,
        out_type=jax.ShapeDtypeStruct((NC, n, d), jnp.float32),
        mesh=mesh,
        scratch_types=[
            pltpu.VMEM((2, CH), jnp.int32),
            pltpu.VMEM((2, CH), jnp.int32),
            pltpu.VMEM((nit, CH), jnp.int32),
            pltpu.VMEM((CH, d), jnp.float32),
            pltpu.VMEM((CH, d), jnp.float32),
            pltpu.VMEM_SHARED((n, d), jnp.float32),
            pltpu.SemaphoreType.DMA,
            pltpu.SemaphoreType.DMA,
            pltpu.SemaphoreType.DMA,
            pltpu.SemaphoreType.DMA,
            pltpu.SemaphoreType.DMA,
            pltpu.SemaphoreType.DMA,
        ],
    )
    def body(h_hbm, pk_hbm, dst_hbm, z_hbm, out_hbm,
             pk0, pk1, dvm, r0, r1, acc, ps0, ps1, gs0, gs1, ss0, ss1):
        c = lax.axis_index("c")
        s = lax.axis_index("s")
        wid = c * NS + s
        pltpu.sync_copy(z_hbm.at[pl.ds(s * slab, slab)],
                        acc.at[pl.ds(s * slab, slab)])

        @pl.when(s == NS - 1)
        def _():
            pltpu.sync_copy(z_hbm.at[pl.ds(NS * slab, tail)],
                            acc.at[pl.ds(NS * slab, tail)])

        plsc.subcore_barrier()

        pltpu.sync_copy(dst_hbm.at[wid], dvm)
        pltpu.sync_copy(pk_hbm.at[wid, 0], pk0)
        pltpu.async_copy(h_hbm.at[pk0.at[0]], r0, gs0)
        pltpu.async_copy(pk_hbm.at[wid, 1], pk1, ps1)

        def process(it, pkc, psem, rows, gsem, ssem,
                    opk, opsem, orows, ogsem, ossem):
            pltpu.make_async_copy(h_hbm.at[pkc.at[0]], rows, gsem).wait()

            @pl.when(it < nit - 1)
            def _():
                pltpu.make_async_copy(pk_hbm.at[wid, it + 1], opk,
                                      opsem).wait()

                @pl.when(it >= 1)
                def _():
                    pltpu.make_async_copy(orows, acc.at[dvm.at[0]],
                                          ossem).wait()

                pltpu.async_copy(h_hbm.at[opk.at[0]], orows, ogsem)

            def scale(g, carry):
                wbits = pkc[1, pl.ds(g * 16, 16)]
                w16 = lax.bitcast_convert_type(wbits, jnp.float32)
                for i in range(16):
                    e_i = g * 16 + i
                    w = w16[i]
                    for j in range(d // 16):
                        sl = pl.ds(j * 16, 16)
                        rows[e_i, sl] = rows[e_i, sl] * w
                return carry

            lax.fori_loop(0, CH // 16, scale, 0)

            pltpu.async_copy(rows, acc.at[dvm.at[it]], ssem, add=True)

            @pl.when(it < nit - 2)
            def _():
                pltpu.async_copy(pk_hbm.at[wid, it + 2], pkc, psem)

        def process_even(it):
            process(it, pk0, ps0, r0, gs0, ss0, pk1, ps1, r1, gs1, ss1)

        def process_odd(it):
            process(it, pk1, ps1, r1, gs1, ss1, pk0, ps0, r0, gs0, ss0)

        def pair(it2, carry):
            process_even(it2 * 2)
            process_odd(it2 * 2 + 1)
            return carry

        lax.fori_loop(0, nit // 2, pair, 0)
        if nit % 2:
            process_even(nit - 1)
        if (nit - 2) % 2 == 0:
            pltpu.make_async_copy(r0, acc.at[dvm.at[0]], ss0).wait()
            pltpu.make_async_copy(r1, acc.at[dvm.at[0]], ss1).wait()
        else:
            pltpu.make_async_copy(r1, acc.at[dvm.at[0]], ss1).wait()
            pltpu.make_async_copy(r0, acc.at[dvm.at[0]], ss0).wait()
        plsc.subcore_barrier()
        pltpu.sync_copy(acc.at[pl.ds(s * slab, slab)],
                        out_hbm.at[c, pl.ds(s * slab, slab)])

        @pl.when(s == NS - 1)
        def _():
            pltpu.sync_copy(acc.at[pl.ds(NS * slab, tail)],
                            out_hbm.at[c, pl.ds(NS * slab, tail)])

    return body(h, pk, dst3, zeros)


def _linear_relu_tc(p, W, b):
    k, n, d = p.shape
    br = 1000

    def body(p_ref, w_ref, b_ref, o_ref):
        xs = jnp.sum(p_ref[...], axis=0)
        y = jnp.dot(xs, w_ref[...], preferred_element_type=jnp.float32)
        o_ref[...] = jnp.maximum(y + b_ref[...], 0.0)

    return pl.pallas_call(
        body,
        grid=(n // br,),
        in_specs=[
            pl.BlockSpec((k, br, d), lambda i: (0, i, 0)),
            pl.BlockSpec((d, d), lambda i: (0, 0)),
            pl.BlockSpec((1, d), lambda i: (0, 0)),
        ],
        out_specs=pl.BlockSpec((br, d), lambda i: (i, 0)),
        out_shape=jax.ShapeDtypeStruct((n, d), jnp.float32),
    )(p, W, b.reshape(1, d))


def _sum_partials_tc(p):
    k, n, d = p.shape
    br = 1000

    def body(p_ref, o_ref):
        o_ref[...] = jnp.sum(p_ref[...], axis=0)

    return pl.pallas_call(
        body,
        grid=(n // br,),
        in_specs=[pl.BlockSpec((k, br, d), lambda i: (0, i, 0))],
        out_specs=pl.BlockSpec((br, d), lambda i: (i, 0)),
        out_shape=jax.ShapeDtypeStruct((n, d), jnp.float32),
    )(p)


def kernel(x, edge_index, edge_weight, W1, b1, W2, b2):
    e = edge_weight.shape[0]
    epw_pad = -(-e // (NW * CH)) * CH
    nit = epw_pad // CH
    e_pad = NW * epw_pad
    src = jnp.pad(edge_index[0], (0, e_pad - e))
    dst3 = jnp.pad(edge_index[1], (0, e_pad - e)).reshape(NW, nit, CH)
    ewb = jnp.pad(edge_weight, (0, e_pad - e)).view(jnp.int32)
    pk = jnp.stack([src, ewb], axis=1).reshape(NW, nit, CH, 2)
    pk = jnp.swapaxes(pk, 2, 3)
    zeros = jnp.zeros(x.shape, jnp.float32)
    h1 = _linear_relu_tc(x[None], W1, b1)
    p1 = _sc_gather_scatter(h1, pk, dst3, zeros)
    h2 = _linear_relu_tc(p1, W2, b2)
    p2 = _sc_gather_scatter(h2, pk, dst3, zeros)
    return _sum_partials_tc(p2)
